# baseline (speedup 1.0000x reference)
import jax
import jax.numpy as jnp
from jax.experimental import pallas as pl

_B, _N, _D, _H = 128, 256, 128, 512


def _fused(h_ref, m_ref, wc_ref, bc_ref, wf_ref, bf_ref, out_ref):
    hb = h_ref[0]
    mb = m_ref[0]
    deg = jnp.sum(mb, axis=1, keepdims=True)
    deg_c = jnp.maximum(deg, 1.0)
    mh = jnp.dot(mb, hb, preferred_element_type=jnp.float32)
    s = jnp.dot(mh, wc_ref[...], preferred_element_type=jnp.float32)
    s = s + deg * bc_ref[...]
    a = jnp.maximum(s / deg_c, 0.0)
    y = jnp.dot(a, wf_ref[...], preferred_element_type=jnp.float32)
    out_ref[0] = y + bf_ref[...]


def kernel(h, mask, W_conv, b_conv, W_ff, b_ff):
    B, N, D = h.shape
    H = W_conv.shape[1]
    return pl.pallas_call(
        _fused,
        grid=(B,),
        in_specs=[
            pl.BlockSpec((1, N, D), lambda b: (b, 0, 0)),
            pl.BlockSpec((1, N, N), lambda b: (b, 0, 0)),
            pl.BlockSpec((D, H), lambda b: (0, 0)),
            pl.BlockSpec((1, H), lambda b: (0, 0)),
            pl.BlockSpec((H, D), lambda b: (0, 0)),
            pl.BlockSpec((1, D), lambda b: (0, 0)),
        ],
        out_specs=pl.BlockSpec((1, N, D), lambda b: (b, 0, 0)),
        out_shape=jax.ShapeDtypeStruct((B, N, D), jnp.float32),
    )(h, mask, W_conv, b_conv.reshape(1, H), W_ff, b_ff.reshape(1, D))


# speedup vs baseline: 2.2921x; 2.2921x over previous
import jax
import jax.numpy as jnp
from jax.experimental import pallas as pl

_BB = 4


def _fused(h_ref, m_ref, wc_ref, bc_ref, wf_ref, bf_ref, out_ref):
    n = m_ref.shape[2]
    hs = h_ref[0]
    mh_parts = []
    deg_parts = []
    for bb in range(_BB):
        mb = m_ref[0, bb]
        deg_parts.append(jnp.sum(mb, axis=1, keepdims=True))
        mh_parts.append(
            jnp.dot(mb, hs[bb * n:(bb + 1) * n, :],
                    preferred_element_type=jnp.float32))
    mh = jnp.concatenate(mh_parts, axis=0)
    deg = jnp.concatenate(deg_parts, axis=0)
    deg_c = jnp.maximum(deg, 1.0)
    s = jnp.dot(mh, wc_ref[...], preferred_element_type=jnp.float32)
    s = s + deg * bc_ref[...]
    a = jnp.maximum(s / deg_c, 0.0)
    y = jnp.dot(a, wf_ref[...], preferred_element_type=jnp.float32)
    out_ref[0] = y + bf_ref[...]


def kernel(h, mask, W_conv, b_conv, W_ff, b_ff):
    B, N, D = h.shape
    H = W_conv.shape[1]
    G = B // _BB
    h2 = h.reshape(G, _BB * N, D)
    m2 = mask.reshape(G, _BB, N, N)
    out = pl.pallas_call(
        _fused,
        grid=(G,),
        in_specs=[
            pl.BlockSpec((1, _BB * N, D), lambda b: (b, 0, 0)),
            pl.BlockSpec((1, _BB, N, N), lambda b: (b, 0, 0, 0)),
            pl.BlockSpec((D, H), lambda b: (0, 0)),
            pl.BlockSpec((1, H), lambda b: (0, 0)),
            pl.BlockSpec((H, D), lambda b: (0, 0)),
            pl.BlockSpec((1, D), lambda b: (0, 0)),
        ],
        out_specs=pl.BlockSpec((1, _BB * N, D), lambda b: (b, 0, 0)),
        out_shape=jax.ShapeDtypeStruct((G, _BB * N, D), jnp.float32),
    )(h2, m2, W_conv, b_conv.reshape(1, H), W_ff, b_ff.reshape(1, D))
    return out.reshape(B, N, D)


# speedup vs baseline: 2.9868x; 1.3031x over previous
import jax
import jax.numpy as jnp
from jax.experimental import pallas as pl

_BB = 8


def _fused(h_ref, m_ref, wc_ref, bc_ref, wf_ref, bf_ref, out_ref):
    n = m_ref.shape[2]
    hs = h_ref[0]
    mh_parts = []
    deg_parts = []
    for bb in range(_BB):
        mb = m_ref[0, bb]
        deg_parts.append(jnp.sum(mb, axis=1, keepdims=True))
        mh_parts.append(
            jnp.dot(mb, hs[bb * n:(bb + 1) * n, :],
                    preferred_element_type=jnp.float32))
    mh = jnp.concatenate(mh_parts, axis=0)
    deg = jnp.concatenate(deg_parts, axis=0)
    deg_c = jnp.maximum(deg, 1.0)
    s = jnp.dot(mh, wc_ref[...], preferred_element_type=jnp.float32)
    s = s + deg * bc_ref[...]
    a = jnp.maximum(s / deg_c, 0.0)
    y = jnp.dot(a, wf_ref[...], preferred_element_type=jnp.float32)
    out_ref[0] = y + bf_ref[...]


def kernel(h, mask, W_conv, b_conv, W_ff, b_ff):
    B, N, D = h.shape
    H = W_conv.shape[1]
    G = B // _BB
    h2 = h.reshape(G, _BB * N, D)
    m2 = mask.reshape(G, _BB, N, N)
    out = pl.pallas_call(
        _fused,
        grid=(G,),
        in_specs=[
            pl.BlockSpec((1, _BB * N, D), lambda b: (b, 0, 0)),
            pl.BlockSpec((1, _BB, N, N), lambda b: (b, 0, 0, 0)),
            pl.BlockSpec((D, H), lambda b: (0, 0)),
            pl.BlockSpec((1, H), lambda b: (0, 0)),
            pl.BlockSpec((H, D), lambda b: (0, 0)),
            pl.BlockSpec((1, D), lambda b: (0, 0)),
        ],
        out_specs=pl.BlockSpec((1, _BB * N, D), lambda b: (b, 0, 0)),
        out_shape=jax.ShapeDtypeStruct((G, _BB * N, D), jnp.float32),
    )(h2, m2, W_conv, b_conv.reshape(1, H), W_ff, b_ff.reshape(1, D))
    return out.reshape(B, N, D)


# speedup vs baseline: 3.3821x; 1.1324x over previous
import jax
import jax.numpy as jnp
from jax.experimental import pallas as pl

_BB = 16


def _fused(h_ref, m_ref, wc_ref, bc_ref, wf_ref, bf_ref, out_ref):
    n = m_ref.shape[2]
    hs = h_ref[0]
    mh_parts = []
    deg_parts = []
    for bb in range(_BB):
        mb = m_ref[0, bb]
        deg_parts.append(jnp.sum(mb, axis=1, keepdims=True))
        mh_parts.append(
            jnp.dot(mb, hs[bb * n:(bb + 1) * n, :],
                    preferred_element_type=jnp.float32))
    mh = jnp.concatenate(mh_parts, axis=0)
    deg = jnp.concatenate(deg_parts, axis=0)
    deg_c = jnp.maximum(deg, 1.0)
    s = jnp.dot(mh, wc_ref[...], preferred_element_type=jnp.float32)
    s = s + deg * bc_ref[...]
    a = jnp.maximum(s / deg_c, 0.0)
    y = jnp.dot(a, wf_ref[...], preferred_element_type=jnp.float32)
    out_ref[0] = y + bf_ref[...]


def kernel(h, mask, W_conv, b_conv, W_ff, b_ff):
    B, N, D = h.shape
    H = W_conv.shape[1]
    G = B // _BB
    h2 = h.reshape(G, _BB * N, D)
    m2 = mask.reshape(G, _BB, N, N)
    out = pl.pallas_call(
        _fused,
        grid=(G,),
        in_specs=[
            pl.BlockSpec((1, _BB * N, D), lambda b: (b, 0, 0)),
            pl.BlockSpec((1, _BB, N, N), lambda b: (b, 0, 0, 0)),
            pl.BlockSpec((D, H), lambda b: (0, 0)),
            pl.BlockSpec((1, H), lambda b: (0, 0)),
            pl.BlockSpec((H, D), lambda b: (0, 0)),
            pl.BlockSpec((1, D), lambda b: (0, 0)),
        ],
        out_specs=pl.BlockSpec((1, _BB * N, D), lambda b: (b, 0, 0)),
        out_shape=jax.ShapeDtypeStruct((G, _BB * N, D), jnp.float32),
    )(h2, m2, W_conv, b_conv.reshape(1, H), W_ff, b_ff.reshape(1, D))
    return out.reshape(B, N, D)


# speedup vs baseline: 3.4184x; 1.0107x over previous
import jax
import jax.numpy as jnp
from jax.experimental import pallas as pl

_BB = 32


def _fused(h_ref, m_ref, wc_ref, bc_ref, wf_ref, bf_ref, out_ref):
    n = m_ref.shape[2]
    hs = h_ref[0]
    mh_parts = []
    deg_parts = []
    for bb in range(_BB):
        mb = m_ref[0, bb]
        deg_parts.append(jnp.sum(mb, axis=1, keepdims=True))
        mh_parts.append(
            jnp.dot(mb, hs[bb * n:(bb + 1) * n, :],
                    preferred_element_type=jnp.float32))
    mh = jnp.concatenate(mh_parts, axis=0)
    deg = jnp.concatenate(deg_parts, axis=0)
    deg_c = jnp.maximum(deg, 1.0)
    s = jnp.dot(mh, wc_ref[...], preferred_element_type=jnp.float32)
    s = s + deg * bc_ref[...]
    a = jnp.maximum(s / deg_c, 0.0)
    y = jnp.dot(a, wf_ref[...], preferred_element_type=jnp.float32)
    out_ref[0] = y + bf_ref[...]


def kernel(h, mask, W_conv, b_conv, W_ff, b_ff):
    B, N, D = h.shape
    H = W_conv.shape[1]
    G = B // _BB
    h2 = h.reshape(G, _BB * N, D)
    m2 = mask.reshape(G, _BB, N, N)
    out = pl.pallas_call(
        _fused,
        grid=(G,),
        in_specs=[
            pl.BlockSpec((1, _BB * N, D), lambda b: (b, 0, 0)),
            pl.BlockSpec((1, _BB, N, N), lambda b: (b, 0, 0, 0)),
            pl.BlockSpec((D, H), lambda b: (0, 0)),
            pl.BlockSpec((1, H), lambda b: (0, 0)),
            pl.BlockSpec((H, D), lambda b: (0, 0)),
            pl.BlockSpec((1, D), lambda b: (0, 0)),
        ],
        out_specs=pl.BlockSpec((1, _BB * N, D), lambda b: (b, 0, 0)),
        out_shape=jax.ShapeDtypeStruct((G, _BB * N, D), jnp.float32),
    )(h2, m2, W_conv, b_conv.reshape(1, H), W_ff, b_ff.reshape(1, D))
    return out.reshape(B, N, D)
